# Initial kernel scaffold; baseline (speedup 1.0000x reference)
import jax
import jax.numpy as jnp
from jax.experimental import pallas as pl


def kernel(word_feat, phrase_feat, rel_feat, rel_conn_mat, word_to_graph_conn, w_ih_f, w_hh_f, b_ih_f, b_hh_f, w_ih_b, w_hh_b, b_ih_b, b_hh_b, w2p_w_W, w2p_w_b, w2p_p_W, w2p_p_b, w2p_t_W, w2p_t_b, rel_emb_W, rel_emb_b, r2p_a1_W, r2p_a1_b, r2p_a2_W, r2p_a2_b, r2p_t_W, r2p_t_b):
    raise NotImplementedError("write your pallas kernel here")


# speedup vs baseline: 3.7570x; 3.7570x over previous
import functools

import jax
import jax.numpy as jnp
from jax import lax
from jax.experimental import pallas as pl
from jax.experimental.pallas import tpu as pltpu

H = 1024
HH = 512
NWORD = 256
NPHR = 2048
NREL = 4096


def _sig(x):
    return 1.0 / (1.0 + jnp.exp(-x))


def _leaky(x):
    return jnp.where(x >= 0, x, 0.01 * x)


def _gru_body(word_ref, wih_f_ref, wih_b_ref, whh_f_ref, whh_b_ref,
              bih_f_ref, bih_b_ref, bhh_f_ref, bhh_b_ref, lc_ref,
              xf_ref, xb_ref):
    word = word_ref[...]
    xf_ref[...] = lax.dot_general(word, wih_f_ref[...],
                                  (((1,), (1,)), ((), ()))) + bih_f_ref[...]
    xb_ref[...] = lax.dot_general(word, wih_b_ref[...],
                                  (((1,), (1,)), ((), ()))) + bih_b_ref[...]
    whh_f = whh_f_ref[...]
    whh_b = whh_b_ref[...]
    bhh_f = bhh_f_ref[...]
    bhh_b = bhh_b_ref[...]

    def gru_step(gi, h, whh, bhh):
        gh = lax.dot_general(h, whh, (((1,), (1,)), ((), ()))) + bhh
        i_r = gi[:, :HH]
        i_z = gi[:, HH:2 * HH]
        i_n = gi[:, 2 * HH:]
        h_r = gh[:, :HH]
        h_z = gh[:, HH:2 * HH]
        h_n = gh[:, 2 * HH:]
        r = _sig(i_r + h_r)
        z = _sig(i_z + h_z)
        n = jnp.tanh(i_n + r * h_n)
        return (1.0 - z) * n + z * h

    def step(t, carry):
        h_f, h_b, h_f0, h_b0 = carry
        gi_f = xf_ref[pl.ds(t, 1), :]
        gi_b = xb_ref[pl.ds(NWORD - 1 - t, 1), :]
        h_f = gru_step(gi_f, h_f, whh_f, bhh_f)
        h_b = gru_step(gi_b, h_b, whh_b, bhh_b)
        first = t == 0
        h_f0 = jnp.where(first, h_f, h_f0)
        h_b0 = jnp.where(first, h_b, h_b0)
        return (h_f, h_b, h_f0, h_b0)

    z0 = jnp.zeros((1, HH), jnp.float32)
    h_f, h_b, h_f0, h_b0 = lax.fori_loop(0, NWORD, step, (z0, z0, z0, z0))
    lc_ref[...] = jnp.concatenate([h_f0, h_b, h_f, h_b0], axis=1)


def _gru(word_feat, w_ih_f, w_ih_b, w_hh_f, w_hh_b,
         b_ih_f, b_ih_b, b_hh_f, b_hh_b):
    return pl.pallas_call(
        _gru_body,
        out_shape=jax.ShapeDtypeStruct((1, 2 * H), jnp.float32),
        scratch_shapes=[
            pltpu.VMEM((NWORD, 3 * HH), jnp.float32),
            pltpu.VMEM((NWORD, 3 * HH), jnp.float32),
        ],
    )(word_feat, w_ih_f, w_ih_b, w_hh_f, w_hh_b,
      b_ih_f, b_ih_b, b_hh_f, b_hh_b)


def _mm_body(x_ref, w_ref, b_ref, o_ref, *, act):
    acc = jnp.dot(x_ref[...], w_ref[...],
                  preferred_element_type=jnp.float32) + b_ref[...]
    if act == "leaky":
        acc = _leaky(acc)
    o_ref[...] = acc


def _mm(x, w, b, act=None, bm=256):
    m, k = x.shape
    n = w.shape[1]
    bm = min(bm, m)
    b2 = b.reshape(1, n)
    return pl.pallas_call(
        functools.partial(_mm_body, act=act),
        grid=(m // bm,),
        in_specs=[
            pl.BlockSpec((bm, k), lambda i: (i, 0)),
            pl.BlockSpec((k, n), lambda i: (0, 0)),
            pl.BlockSpec((1, n), lambda i: (0, 0)),
        ],
        out_specs=pl.BlockSpec((bm, n), lambda i: (i, 0)),
        out_shape=jax.ShapeDtypeStruct((m, n), jnp.float32),
    )(x, w, b2)


def _w2p_body(phr_ref, word_ref, conn_ref, wp_ref, ww_ref, wt_ref,
              bp_ref, bw_ref, bt_ref, out_ref, pw_ref):
    i = pl.program_id(0)

    @pl.when(i == 0)
    def _():
        pw_ref[...] = _leaky(jnp.dot(word_ref[...], ww_ref[...],
                                     preferred_element_type=jnp.float32)
                             + bw_ref[...])

    phr = phr_ref[...]
    pp = _leaky(jnp.dot(phr, wp_ref[...],
                        preferred_element_type=jnp.float32) + bp_ref[...])
    logits = lax.dot_general(pp, pw_ref[...],
                             (((1,), (1,)), ((), ()))) * (1.0 / (H ** 0.5))
    m = jnp.max(logits, axis=-1, keepdims=True)
    e = jnp.exp(logits - m)
    me = e * conn_ref[...].astype(jnp.float32)
    a = me / (jnp.sum(me, axis=-1, keepdims=True) + 1e-6)
    ctx = jnp.dot(a, word_ref[...], preferred_element_type=jnp.float32)
    out_ref[...] = phr + jnp.dot(ctx, wt_ref[...],
                                 preferred_element_type=jnp.float32) + bt_ref[...]


def _w2p(phrase_feat, word_feat, conn, wp, ww, wt, bp, bw, bt, bm=256):
    return pl.pallas_call(
        _w2p_body,
        grid=(NPHR // bm,),
        in_specs=[
            pl.BlockSpec((bm, H), lambda i: (i, 0)),
            pl.BlockSpec((NWORD, H), lambda i: (0, 0)),
            pl.BlockSpec((bm, NWORD), lambda i: (i, 0)),
            pl.BlockSpec((H, H), lambda i: (0, 0)),
            pl.BlockSpec((H, H), lambda i: (0, 0)),
            pl.BlockSpec((H, H), lambda i: (0, 0)),
            pl.BlockSpec((1, H), lambda i: (0, 0)),
            pl.BlockSpec((1, H), lambda i: (0, 0)),
            pl.BlockSpec((1, H), lambda i: (0, 0)),
        ],
        out_specs=pl.BlockSpec((bm, H), lambda i: (i, 0)),
        out_shape=jax.ShapeDtypeStruct((NPHR, H), jnp.float32),
        scratch_shapes=[pltpu.VMEM((NWORD, H), jnp.float32)],
    )(phrase_feat, word_feat, conn, wp, ww, wt,
      bp.reshape(1, H), bw.reshape(1, H), bt.reshape(1, H))


def _onehot(idx_col, n):
    cols = lax.broadcasted_iota(jnp.int32, (idx_col.shape[0], n), 1)
    return (idx_col == cols).astype(jnp.float32)


def _relemb_body(s_ref, o_ref, r2_ref, p0_ref, p1_ref, out_ref):
    oh_s = _onehot(s_ref[...], NPHR)
    oh_o = _onehot(o_ref[...], NPHR)
    acc = jnp.dot(oh_s, p0_ref[...], preferred_element_type=jnp.float32)
    acc += jnp.dot(oh_o, p1_ref[...], preferred_element_type=jnp.float32)
    out_ref[...] = acc + r2_ref[...]


def _relemb(subj2, obj2, r2, p0, p1, bm=512):
    return pl.pallas_call(
        _relemb_body,
        grid=(NREL // bm,),
        in_specs=[
            pl.BlockSpec((bm, 1), lambda i: (i, 0)),
            pl.BlockSpec((bm, 1), lambda i: (i, 0)),
            pl.BlockSpec((bm, H), lambda i: (i, 0)),
            pl.BlockSpec((NPHR, H), lambda i: (0, 0)),
            pl.BlockSpec((NPHR, H), lambda i: (0, 0)),
        ],
        out_specs=pl.BlockSpec((bm, H), lambda i: (i, 0)),
        out_shape=jax.ShapeDtypeStruct((NREL, H), jnp.float32),
    )(subj2, obj2, r2, p0, p1)


def _score_body(s_ref, o_ref, ra_ref, pa_ref, a2_ref, a2b_ref,
                vs_ref, vo_ref):
    oh_s = _onehot(s_ref[...], NPHR)
    oh_o = _onehot(o_ref[...], NPHR)
    pa = pa_ref[...]
    ra = ra_ref[...]
    a2 = a2_ref[...]
    g_s = jnp.dot(oh_s, pa, preferred_element_type=jnp.float32)
    g_o = jnp.dot(oh_o, pa, preferred_element_type=jnp.float32)
    vs_ref[...] = jnp.dot(_leaky(g_s + ra), a2,
                          preferred_element_type=jnp.float32) + a2b_ref[...]
    vo_ref[...] = jnp.dot(_leaky(g_o + ra), a2,
                          preferred_element_type=jnp.float32) + a2b_ref[...]


def _score(subj2, obj2, rel_arg, phr_a, a2, a2b, bm=512):
    return pl.pallas_call(
        _score_body,
        grid=(NREL // bm,),
        in_specs=[
            pl.BlockSpec((bm, 1), lambda i: (i, 0)),
            pl.BlockSpec((bm, 1), lambda i: (i, 0)),
            pl.BlockSpec((bm, H), lambda i: (i, 0)),
            pl.BlockSpec((NPHR, H), lambda i: (0, 0)),
            pl.BlockSpec((H, 1), lambda i: (0, 0)),
            pl.BlockSpec((1, 1), lambda i: (0, 0)),
        ],
        out_specs=[
            pl.BlockSpec((bm, 1), lambda i: (i, 0)),
            pl.BlockSpec((bm, 1), lambda i: (i, 0)),
        ],
        out_shape=[
            jax.ShapeDtypeStruct((NREL, 1), jnp.float32),
            jax.ShapeDtypeStruct((NREL, 1), jnp.float32),
        ],
    )(subj2, obj2, rel_arg, phr_a, a2, a2b.reshape(1, 1))


def _att_body(s_ref, o_ref, vs_ref, vo_ref, phr_ref, rel_ref, wt_ref,
              bt_ref, out_ref):
    i = pl.program_id(0)
    bm = phr_ref.shape[0]
    pids = i * bm + lax.broadcasted_iota(jnp.int32, (bm, 1), 0)
    subj = s_ref[...]
    obj = o_ref[...]
    is_o = obj == pids
    is_s = subj == pids
    att = jnp.where(is_o, vo_ref[...],
                    jnp.where(is_s, vs_ref[...], 0.0))
    m = jnp.max(att, axis=-1, keepdims=True)
    e = jnp.exp(att - m)
    me = e * (is_o | is_s).astype(jnp.float32)
    a = me / (jnp.sum(me, axis=-1, keepdims=True) + 1e-6)
    ctx = jnp.dot(a, rel_ref[...], preferred_element_type=jnp.float32)
    out_ref[...] = phr_ref[...] + jnp.dot(
        ctx, wt_ref[...], preferred_element_type=jnp.float32) + bt_ref[...]


def _att(subj_row, obj_row, vs_row, vo_row, upd_phr, upd_rel, wt, bt, bm=256):
    return pl.pallas_call(
        _att_body,
        grid=(NPHR // bm,),
        in_specs=[
            pl.BlockSpec((1, NREL), lambda i: (0, 0)),
            pl.BlockSpec((1, NREL), lambda i: (0, 0)),
            pl.BlockSpec((1, NREL), lambda i: (0, 0)),
            pl.BlockSpec((1, NREL), lambda i: (0, 0)),
            pl.BlockSpec((bm, H), lambda i: (i, 0)),
            pl.BlockSpec((NREL, H), lambda i: (0, 0)),
            pl.BlockSpec((H, H), lambda i: (0, 0)),
            pl.BlockSpec((1, H), lambda i: (0, 0)),
        ],
        out_specs=pl.BlockSpec((bm, H), lambda i: (i, 0)),
        out_shape=jax.ShapeDtypeStruct((NPHR, H), jnp.float32),
    )(subj_row, obj_row, vs_row, vo_row, upd_phr, upd_rel,
      wt, bt.reshape(1, H))


def kernel(word_feat, phrase_feat, rel_feat, rel_conn_mat, word_to_graph_conn,
           w_ih_f, w_hh_f, b_ih_f, b_hh_f, w_ih_b, w_hh_b, b_ih_b, b_hh_b,
           w2p_w_W, w2p_w_b, w2p_p_W, w2p_p_b, w2p_t_W, w2p_t_b,
           rel_emb_W, rel_emb_b, r2p_a1_W, r2p_a1_b, r2p_a2_W, r2p_a2_b,
           r2p_t_W, r2p_t_b):
    subj = rel_conn_mat[0]
    obj = rel_conn_mat[1]
    subj2 = subj.reshape(NREL, 1)
    obj2 = obj.reshape(NREL, 1)

    lc = _gru(word_feat, w_ih_f, w_ih_b, w_hh_f, w_hh_b,
              b_ih_f.reshape(1, -1), b_ih_b.reshape(1, -1),
              b_hh_f.reshape(1, -1), b_hh_b.reshape(1, -1))

    upd_phr = _w2p(phrase_feat, word_feat, word_to_graph_conn,
                   w2p_p_W, w2p_w_W, w2p_t_W, w2p_p_b, w2p_w_b, w2p_t_b)

    wcat = jnp.concatenate([rel_emb_W[:H], rel_emb_W[H:2 * H],
                            r2p_a1_W[:H]], axis=1)
    proj = _mm(upd_phr, wcat, jnp.zeros((3 * H,), jnp.float32))
    p0 = proj[:, :H]
    p1 = proj[:, H:2 * H]
    phr_a = proj[:, 2 * H:]

    r2 = _mm(rel_feat, rel_emb_W[2 * H:], rel_emb_b)
    upd_rel = _relemb(subj2, obj2, r2, p0, p1)

    base = _mm(lc, r2p_a1_W[2 * H:], r2p_a1_b, bm=1)
    rel_arg = _mm(upd_rel, r2p_a1_W[H:2 * H],
                  jnp.zeros((H,), jnp.float32)) + base

    v_s, v_o = _score(subj2, obj2, rel_arg, phr_a, r2p_a2_W, r2p_a2_b)

    out_phr = _att(subj.reshape(1, NREL), obj.reshape(1, NREL),
                   v_s.reshape(1, NREL), v_o.reshape(1, NREL),
                   upd_phr, upd_rel, r2p_t_W, r2p_t_b)
    return (word_feat, out_phr, upd_rel)
